# baseline (speedup 1.0000x reference)
import functools

import jax
import jax.numpy as jnp
from jax import lax
from jax.experimental import pallas as pl
from jax.experimental.pallas import tpu as pltpu
from jax.experimental.pallas import tpu_sc as plsc

D_MODEL = 64
GROUP = 128
NJ = 4
NDT = D_MODEL // 8


@functools.cache
def _build(n_rows: int, n_btiles: int, num_cores: int, num_subcores: int):
    num_workers = num_cores * num_subcores
    jc_per_row = n_btiles // NJ
    n_chunks = n_rows * jc_per_row
    chunks_per_worker = n_chunks // num_workers
    groups_per_worker = chunks_per_worker * NJ
    assert chunks_per_worker * num_workers == n_chunks
    assert chunks_per_worker % 2 == 0

    mesh = plsc.VectorSubcoreMesh(core_axis_name="c", subcore_axis_name="s")

    @functools.partial(
        pl.kernel,
        out_type=(
            jax.ShapeDtypeStruct((n_rows, NDT, n_btiles, 8, GROUP), jnp.float32),
            jax.ShapeDtypeStruct((NJ * D_MODEL, GROUP), jnp.float32),
        ),
        mesh=mesh,
        scratch_types=[
            pltpu.VMEM((groups_per_worker, GROUP), jnp.int32),
            pltpu.VMEM((2, NJ, GROUP, D_MODEL), jnp.float32),
            pltpu.VMEM((NJ * D_MODEL, GROUP), jnp.float32),
            pltpu.SemaphoreType.DMA,
            pltpu.SemaphoreType.DMA,
            pltpu.SemaphoreType.DMA,
        ],
        compiler_params=pltpu.CompilerParams(
            use_tc_tiling_on_sc=False, needs_layout_passes=False
        ),
    )
    def gather_kernel(idx_hbm, table_hbm, out_hbm, dummy_hbm, idx_v, rows_v,
                      rowsT_v, sem_g0, sem_g1, sem_s):
        wid = lax.axis_index("s") * num_cores + lax.axis_index("c")
        c0 = wid * chunks_per_worker
        c_end = c0 + chunks_per_worker
        sems = (sem_g0, sem_g1)
        lane = lax.iota(jnp.int32, 16)

        pltpu.sync_copy(
            idx_hbm.at[pl.ds(c0 * NJ, groups_per_worker)], idx_v
        )

        def fire(c, buf):
            g0 = (c - c0) * NJ
            for j in range(NJ):
                pltpu.async_copy(
                    table_hbm.at[idx_v.at[g0 + j]], rows_v.at[buf, j],
                    sems[buf],
                )

        def wait_gathers(c, buf):
            g0 = (c - c0) * NJ
            for j in range(NJ):
                pltpu.make_async_copy(
                    table_hbm.at[idx_v.at[g0 + j]], rows_v.at[buf, j],
                    sems[buf],
                ).wait()

        def drain_stores():
            pltpu.make_async_copy(dummy_hbm, rowsT_v, sem_s).wait()

        def consume(c, buf):
            r = c // jc_per_row
            j0 = (c % jc_per_row) * NJ

            pgs = [(lane + s) & 15 for s in range(16)]

            @plsc.parallel_loop(0, GROUP // 16)
            def _bb(bb):
                bvec = bb * 16 + lane
                for s in range(16):
                    pg = pgs[s]
                    for j in range(NJ):
                        src = rows_v.at[buf, j]
                        for dg in range(D_MODEL // 16):
                            d0 = dg * 16
                            dst = rowsT_v.at[pl.ds(j * D_MODEL + d0, 16)]
                            v = plsc.load_gather(src, [bvec, d0 + pg])
                            plsc.store_scatter(dst, [pg, bvec], v)

            for dt in range(NDT):
                for j in range(NJ):
                    pltpu.async_copy(
                        rowsT_v.at[pl.ds(j * D_MODEL + dt * 8, 8)],
                        out_hbm.at[r, dt, j0 + j],
                        sem_s,
                    )

        fire(c0, 0)

        @pl.loop(0, chunks_per_worker, step=2)
        def _outer(i):
            c = c0 + i
            for b in range(2):
                cb = c + b

                @pl.when(cb + 1 < c_end)
                def _():
                    fire(cb + 1, 1 - b)

                @pl.when(cb > c0)
                def _():
                    drain_stores()

                wait_gathers(cb, b)
                consume(cb, b)

        drain_stores()

    return gather_kernel


def kernel(x, table):
    batch, max_n = x.shape
    n_btiles = batch // GROUP
    xt = x.T.reshape(max_n * n_btiles, GROUP).astype(jnp.int32)
    info = plsc.get_sparse_core_info()
    f = _build(max_n, n_btiles, info.num_cores, info.num_subcores)
    z, _ = f(xt, table)
    return z.transpose(2, 4, 0, 1, 3).reshape(batch, max_n, D_MODEL)


# speedup vs baseline: 1.4027x; 1.4027x over previous
import functools

import jax
import jax.numpy as jnp
from jax import lax
from jax.experimental import pallas as pl
from jax.experimental.pallas import tpu as pltpu
from jax.experimental.pallas import tpu_sc as plsc

D_MODEL = 64
GROUP = 128
NJ = 4
NDT = D_MODEL // 8


@functools.cache
def _build(n_rows: int, n_btiles: int, num_cores: int, num_subcores: int):
    num_workers = num_cores * num_subcores
    jc_per_row = n_btiles // NJ
    n_chunks = n_rows * jc_per_row
    chunks_per_worker = n_chunks // num_workers
    groups_per_worker = chunks_per_worker * NJ
    assert chunks_per_worker * num_workers == n_chunks
    assert chunks_per_worker % 2 == 0

    mesh = plsc.VectorSubcoreMesh(core_axis_name="c", subcore_axis_name="s")

    @functools.partial(
        pl.kernel,
        out_type=(
            jax.ShapeDtypeStruct((n_rows, NDT, n_btiles, 8, GROUP), jnp.float32),
            jax.ShapeDtypeStruct((NJ * D_MODEL, GROUP), jnp.float32),
        ),
        mesh=mesh,
        scratch_types=[
            pltpu.VMEM((groups_per_worker, GROUP), jnp.int32),
            pltpu.VMEM((2, NJ, GROUP, D_MODEL), jnp.float32),
            pltpu.VMEM((NJ * D_MODEL, GROUP), jnp.float32),
            pltpu.SemaphoreType.DMA,
            pltpu.SemaphoreType.DMA,
            pltpu.SemaphoreType.DMA,
        ],
        compiler_params=pltpu.CompilerParams(
            use_tc_tiling_on_sc=False, needs_layout_passes=False
        ),
    )
    def gather_kernel(idx_hbm, table_hbm, out_hbm, dummy_hbm, idx_v, rows_v,
                      rowsT_v, sem_g0, sem_g1, sem_s):
        wid = lax.axis_index("s") * num_cores + lax.axis_index("c")
        c0 = wid * chunks_per_worker
        c_end = c0 + chunks_per_worker
        sems = (sem_g0, sem_g1)
        lane = lax.iota(jnp.int32, 16)

        pltpu.sync_copy(
            idx_hbm.at[pl.ds(c0 * NJ, groups_per_worker)], idx_v
        )

        def fire(c, buf):
            g0 = (c - c0) * NJ
            for j in range(NJ):
                pltpu.async_copy(
                    table_hbm.at[idx_v.at[g0 + j]], rows_v.at[buf, j],
                    sems[buf],
                )

        def wait_gathers(c, buf):
            g0 = (c - c0) * NJ
            for j in range(NJ):
                pltpu.make_async_copy(
                    table_hbm.at[idx_v.at[g0 + j]], rows_v.at[buf, j],
                    sems[buf],
                ).wait()

        def drain_stores():
            pltpu.make_async_copy(dummy_hbm, rowsT_v, sem_s).wait()

        def consume(c, buf):
            r = c // jc_per_row
            j0 = (c % jc_per_row) * NJ

            @plsc.parallel_loop(0, (GROUP // 16) * 16, unroll=16)
            def _bs(i):
                bb = i >> 4
                bvec = bb * 16 + lane
                pg = (lane + (i & 15)) & 15
                for j in range(NJ):
                    src = rows_v.at[buf, j]
                    for dg in range(D_MODEL // 16):
                        d0 = dg * 16
                        dst = rowsT_v.at[pl.ds(j * D_MODEL + d0, 16)]
                        v = plsc.load_gather(src, [bvec, d0 + pg])
                        plsc.store_scatter(dst, [pg, bvec], v)

            for dt in range(NDT):
                for j in range(NJ):
                    pltpu.async_copy(
                        rowsT_v.at[pl.ds(j * D_MODEL + dt * 8, 8)],
                        out_hbm.at[r, dt, j0 + j],
                        sem_s,
                    )

        fire(c0, 0)

        @pl.loop(0, chunks_per_worker, step=2)
        def _outer(i):
            c = c0 + i
            for b in range(2):
                cb = c + b

                @pl.when(cb + 1 < c_end)
                def _():
                    fire(cb + 1, 1 - b)

                @pl.when(cb > c0)
                def _():
                    drain_stores()

                wait_gathers(cb, b)
                consume(cb, b)

        drain_stores()

    return gather_kernel


def kernel(x, table):
    batch, max_n = x.shape
    n_btiles = batch // GROUP
    xt = x.T.reshape(max_n * n_btiles, GROUP).astype(jnp.int32)
    info = plsc.get_sparse_core_info()
    f = _build(max_n, n_btiles, info.num_cores, info.num_subcores)
    z, _ = f(xt, table)
    return z.transpose(2, 4, 0, 1, 3).reshape(batch, max_n, D_MODEL)


# speedup vs baseline: 2.0099x; 1.4329x over previous
import functools

import jax
import jax.numpy as jnp
from jax import lax
from jax.experimental import pallas as pl
from jax.experimental.pallas import tpu as pltpu
from jax.experimental.pallas import tpu_sc as plsc

D_MODEL = 64
GROUP = 128
NJ = 4
NDT = D_MODEL // 8


@functools.cache
def _build(n_rows: int, n_btiles: int, num_cores: int, num_subcores: int):
    num_workers = num_cores * num_subcores
    jc_per_row = n_btiles // NJ
    n_chunks = n_rows * jc_per_row
    chunks_per_worker = n_chunks // num_workers
    groups_per_worker = chunks_per_worker * NJ
    assert chunks_per_worker * num_workers == n_chunks
    assert chunks_per_worker % 2 == 0

    mesh = plsc.VectorSubcoreMesh(core_axis_name="c", subcore_axis_name="s")

    @functools.partial(
        pl.kernel,
        out_type=(
            jax.ShapeDtypeStruct((n_rows, NDT, n_btiles, 8, GROUP), jnp.float32),
            jax.ShapeDtypeStruct((NJ * D_MODEL, GROUP), jnp.float32),
        ),
        mesh=mesh,
        scratch_types=[
            pltpu.VMEM((groups_per_worker, GROUP), jnp.int32),
            pltpu.VMEM((2, NJ, GROUP, D_MODEL), jnp.float32),
            pltpu.VMEM((NJ * D_MODEL, GROUP), jnp.float32),
            pltpu.SemaphoreType.DMA,
            pltpu.SemaphoreType.DMA,
            pltpu.SemaphoreType.DMA,
        ],
        compiler_params=pltpu.CompilerParams(
            use_tc_tiling_on_sc=False, needs_layout_passes=False
        ),
    )
    def gather_kernel(idx_hbm, table_hbm, out_hbm, dummy_hbm, idx_v, rows_v,
                      rowsT_v, sem_g0, sem_g1, sem_s):
        wid = lax.axis_index("s") * num_cores + lax.axis_index("c")
        c0 = wid * chunks_per_worker
        c_end = c0 + chunks_per_worker
        sems = (sem_g0, sem_g1)
        lane = lax.iota(jnp.int32, 16)

        pltpu.sync_copy(
            idx_hbm.at[pl.ds(c0 * NJ, groups_per_worker)], idx_v
        )

        def fire(c, buf):
            g0 = (c - c0) * NJ
            for j in range(NJ):
                pltpu.async_copy(
                    table_hbm.at[idx_v.at[g0 + j]], rows_v.at[buf, j],
                    sems[buf],
                )

        def wait_gathers(c, buf):
            g0 = (c - c0) * NJ
            for j in range(NJ):
                pltpu.make_async_copy(
                    table_hbm.at[idx_v.at[g0 + j]], rows_v.at[buf, j],
                    sems[buf],
                ).wait()

        def drain_stores():
            pltpu.make_async_copy(dummy_hbm, rowsT_v, sem_s).wait()

        def consume(c, buf):
            r = c // jc_per_row
            j0 = (c % jc_per_row) * NJ

            @plsc.parallel_loop(0, (GROUP // 16) * 16, unroll=8)
            def _bs(i):
                bb = i >> 4
                bvec = bb * 16 + lane
                pg = (lane + (i & 15)) & 15
                for j in range(NJ):
                    src = rows_v.at[buf, j]
                    for dg in range(D_MODEL // 16):
                        d0 = dg * 16
                        dst = rowsT_v.at[pl.ds(j * D_MODEL + d0, 16)]
                        v = plsc.load_gather(src, [bvec, d0 + pg])
                        plsc.store_scatter(dst, [pg, bvec], v)

            for dt in range(NDT):
                for j in range(NJ):
                    pltpu.async_copy(
                        rowsT_v.at[pl.ds(j * D_MODEL + dt * 8, 8)],
                        out_hbm.at[r, dt, j0 + j],
                        sem_s,
                    )

        fire(c0, 0)

        @pl.loop(0, chunks_per_worker, step=2)
        def _outer(i):
            c = c0 + i
            for b in range(2):
                cb = c + b

                @pl.when(cb + 1 < c_end)
                def _():
                    fire(cb + 1, 1 - b)

                @pl.when(cb > c0)
                def _():
                    drain_stores()

                wait_gathers(cb, b)
                consume(cb, b)

        drain_stores()

    return gather_kernel


def kernel(x, table):
    batch, max_n = x.shape
    n_btiles = batch // GROUP
    xt = x.T.reshape(max_n * n_btiles, GROUP).astype(jnp.int32)
    info = plsc.get_sparse_core_info()
    f = _build(max_n, n_btiles, info.num_cores, info.num_subcores)
    z, _ = f(xt, table)
    return z.transpose(2, 4, 0, 1, 3).reshape(batch, max_n, D_MODEL)
